# Initial kernel scaffold; baseline (speedup 1.0000x reference)
import jax
import jax.numpy as jnp
from jax.experimental import pallas as pl


def kernel(logits):
    raise NotImplementedError("write your pallas kernel here")


# speedup vs baseline: 1.1642x; 1.1642x over previous
import functools

import jax
import jax.numpy as jnp
from jax import lax
from jax.experimental import pallas as pl
from jax.experimental.pallas import tpu as pltpu
from jax.experimental.pallas import tpu_sc as plsc

_K = 8
_E = 64
_T = 32768
_NC = 2
_NS = 16
_L = 16
_NW = _NC * _NS
_TPW = _T // _NW
_CHUNK = 256
_NCHUNK = _TPW // _CHUNK


def _router_body(logits_hbm, w_hbm, id_hbm, hist_hbm, in_v, w_stage, id_stage,
                 hist_v, hist_red):
    wid = lax.axis_index("s") * _NC + lax.axis_index("c")
    tok0 = wid * _TPW

    lane = lax.iota(jnp.int32, 16)
    laneoff = lane * _E
    ones = jnp.ones((_L,), jnp.float32)
    neg_inf = jnp.full((_L,), -jnp.inf, jnp.float32)
    zeros_i = jnp.zeros((_L,), jnp.int32)

    for b in range(_L * _E // _L):
        hist_v[pl.ds(b * _L, _L)] = jnp.zeros((_L,), jnp.float32)

    def chunk_body(ci, carry):
        cbase = tok0 + ci * _CHUNK
        pltpu.sync_copy(logits_hbm.at[pl.ds(cbase * _E, _CHUNK * _E)], in_v)

        def group_body(g, carry2):
            rows = g * _L + lane
            rowsE = rows * _E
            rowsK = rows * _K

            def exp_body(e, st):
                ws, ids = st
                ecol = zeros_i + e
                v = plsc.load_gather(in_v, [rowsE + e])
                m = [v > ws[j] for j in range(_K)]
                mx = [jnp.maximum(v, ws[j]) for j in range(_K)]
                inner = [jnp.where(m[j], ecol, ids[j]) for j in range(_K)]
                nw = [mx[0]] + [jnp.where(m[j - 1], ws[j - 1], mx[j])
                                for j in range(1, _K)]
                ni = [inner[0]] + [jnp.where(m[j - 1], ids[j - 1], inner[j])
                                   for j in range(1, _K)]
                return tuple(nw), tuple(ni)

            ws, ids = lax.fori_loop(
                0, _E, exp_body,
                (tuple([neg_inf] * _K), tuple([zeros_i] * _K)),
                unroll=4)

            es = [jnp.exp(ws[j] - ws[0]) for j in range(_K)]
            s = es[0]
            for j in range(1, _K):
                s = s + es[j]
            r = 1.0 / s

            for j in range(_K):
                plsc.store_scatter(w_stage, [rowsK + j], es[j] * r)
                plsc.store_scatter(id_stage, [rowsK + j], ids[j])
                plsc.addupdate_scatter(hist_v, [ids[j] + laneoff], ones)
            return carry2

        lax.fori_loop(0, _CHUNK // _L, group_body, 0)
        pltpu.sync_copy(w_stage, w_hbm.at[pl.ds(cbase * _K, _CHUNK * _K)])
        pltpu.sync_copy(id_stage, id_hbm.at[pl.ds(cbase * _K, _CHUNK * _K)])
        return carry

    lax.fori_loop(0, _NCHUNK, chunk_body, 0)

    for c in range(_E // _L):
        acc = hist_v[pl.ds(c * _L, _L)]
        for rr in range(1, _L):
            acc = acc + hist_v[pl.ds(rr * _E + c * _L, _L)]
        hist_red[pl.ds(c * _L, _L)] = acc
    pltpu.sync_copy(hist_red, hist_hbm.at[wid])


_router = functools.partial(
    pl.kernel,
    out_type=(
        jax.ShapeDtypeStruct((_T * _K,), jnp.float32),
        jax.ShapeDtypeStruct((_T * _K,), jnp.int32),
        jax.ShapeDtypeStruct((_NW, _E), jnp.float32),
    ),
    mesh=plsc.VectorSubcoreMesh(core_axis_name="c", subcore_axis_name="s"),
    compiler_params=pltpu.CompilerParams(needs_layout_passes=False),
    scratch_types=[
        pltpu.VMEM((_CHUNK * _E,), jnp.float32),
        pltpu.VMEM((_CHUNK * _K,), jnp.float32),
        pltpu.VMEM((_CHUNK * _K,), jnp.int32),
        pltpu.VMEM((_L * _E,), jnp.float32),
        pltpu.VMEM((_E,), jnp.float32),
    ],
)(_router_body)


def _hist_reduce_body(p_ref, o_ref):
    o_ref[...] = jnp.sum(p_ref[...], axis=0, keepdims=True)


def _hist_reduce(partials):
    out = pl.pallas_call(
        _hist_reduce_body,
        out_shape=jax.ShapeDtypeStruct((1, _E), jnp.float32),
    )(partials)
    return out.reshape(_E)


@jax.jit
def kernel(logits):
    topk_w, topk_ids, partials = _router(logits.reshape(-1))
    tokens_per_expert = _hist_reduce(partials)
    return (logits, topk_w.reshape(_T, _K), topk_ids.reshape(_T, _K),
            tokens_per_expert)


# speedup vs baseline: 1.2956x; 1.1129x over previous
import functools

import jax
import jax.numpy as jnp
from jax import lax
from jax.experimental import pallas as pl
from jax.experimental.pallas import tpu as pltpu
from jax.experimental.pallas import tpu_sc as plsc

_K = 8
_E = 64
_T = 32768
_NC = 2
_NS = 16
_L = 16
_NW = _NC * _NS
_TPW = _T // _NW
_CHUNK = 256
_NCHUNK = _TPW // _CHUNK


def _router_body(logits_hbm, w_hbm, id_hbm, hist_hbm, in_v, w_stage, id_stage,
                 hist_v, hist_red):
    wid = lax.axis_index("s") * _NC + lax.axis_index("c")
    tok0 = wid * _TPW

    lane = lax.iota(jnp.int32, 16)
    ones = jnp.ones((_L,), jnp.float32)
    neg_inf = jnp.full((_L,), -jnp.inf, jnp.float32)
    zeros_i = jnp.zeros((_L,), jnp.int32)

    for b in range(_L):
        for c in range(_E // _L):
            hist_v[b, pl.ds(c * _L, _L)] = jnp.zeros((_L,), jnp.float32)

    def chunk_body(ci, carry):
        cbase = tok0 + ci * _CHUNK
        pltpu.sync_copy(logits_hbm.at[pl.ds(cbase, _CHUNK)], in_v)

        def group_body(g, carry2):
            rows = g * _L + lane

            def exp_body(e, st):
                ws, ids = st
                ecol = zeros_i + e
                v = plsc.load_gather(in_v, [rows, ecol])
                m = [v > ws[j] for j in range(_K)]
                mx = [jnp.maximum(v, ws[j]) for j in range(_K)]
                inner = [jnp.where(m[j], ecol, ids[j]) for j in range(_K)]
                nw = [mx[0]] + [jnp.where(m[j - 1], ws[j - 1], mx[j])
                                for j in range(1, _K)]
                ni = [inner[0]] + [jnp.where(m[j - 1], ids[j - 1], inner[j])
                                   for j in range(1, _K)]
                return tuple(nw), tuple(ni)

            ws, ids = lax.fori_loop(
                0, _E, exp_body,
                (tuple([neg_inf] * _K), tuple([zeros_i] * _K)),
                unroll=4)

            es = [jnp.exp(ws[j] - ws[0]) for j in range(_K)]
            s = es[0]
            for j in range(1, _K):
                s = s + es[j]
            r = 1.0 / s

            for j in range(_K):
                col = zeros_i + j
                plsc.store_scatter(w_stage, [rows, col], es[j] * r)
                plsc.store_scatter(id_stage, [rows, col], ids[j])
                plsc.addupdate_scatter(hist_v, [lane, ids[j]], ones)
            return carry2

        lax.fori_loop(0, _CHUNK // _L, group_body, 0)
        pltpu.sync_copy(w_stage, w_hbm.at[pl.ds(cbase, _CHUNK)])
        pltpu.sync_copy(id_stage, id_hbm.at[pl.ds(cbase, _CHUNK)])
        return carry

    lax.fori_loop(0, _NCHUNK, chunk_body, 0)

    for c in range(_E // _L):
        acc = hist_v[0, pl.ds(c * _L, _L)]
        for rr in range(1, _L):
            acc = acc + hist_v[rr, pl.ds(c * _L, _L)]
        hist_red[pl.ds(c * _L, _L)] = acc
    pltpu.sync_copy(hist_red, hist_hbm.at[wid])


_router = functools.partial(
    pl.kernel,
    out_type=(
        jax.ShapeDtypeStruct((_T, _K), jnp.float32),
        jax.ShapeDtypeStruct((_T, _K), jnp.int32),
        jax.ShapeDtypeStruct((_NW, _E), jnp.float32),
    ),
    mesh=plsc.VectorSubcoreMesh(core_axis_name="c", subcore_axis_name="s"),
    compiler_params=pltpu.CompilerParams(needs_layout_passes=False),
    scratch_types=[
        pltpu.VMEM((_CHUNK, _E), jnp.float32),
        pltpu.VMEM((_CHUNK, _K), jnp.float32),
        pltpu.VMEM((_CHUNK, _K), jnp.int32),
        pltpu.VMEM((_L, _E), jnp.float32),
        pltpu.VMEM((_E,), jnp.float32),
    ],
)(_router_body)


def _hist_reduce_body(p_ref, o_ref):
    o_ref[...] = jnp.sum(p_ref[...], axis=0, keepdims=True)


def _hist_reduce(partials):
    out = pl.pallas_call(
        _hist_reduce_body,
        out_shape=jax.ShapeDtypeStruct((1, _E), jnp.float32),
    )(partials)
    return out.reshape(_E)


@jax.jit
def kernel(logits):
    topk_w, topk_ids, partials = _router(logits)
    tokens_per_expert = _hist_reduce(partials)
    return (logits, topk_w, topk_ids, tokens_per_expert)


# speedup vs baseline: 1.5273x; 1.1788x over previous
import functools

import jax
import jax.numpy as jnp
from jax import lax
from jax.experimental import pallas as pl
from jax.experimental.pallas import tpu as pltpu
from jax.experimental.pallas import tpu_sc as plsc

_K = 8
_E = 64
_T = 32768
_NC = 2
_NS = 16
_L = 16
_NW = _NC * _NS
_TPW = _T // _NW
_CHUNK = 256
_NCHUNK = _TPW // _CHUNK


def _router_body(logits_hbm, w_hbm, id_hbm, hist_hbm, in_v, w_stage, id_stage,
                 hist_v, hist_red):
    wid = lax.axis_index("s") * _NC + lax.axis_index("c")
    tok0 = wid * _TPW

    lane = lax.iota(jnp.int32, 16)
    ones = jnp.ones((_L,), jnp.float32)
    neg_inf = jnp.full((_L,), -jnp.inf, jnp.float32)
    zeros_i = jnp.zeros((_L,), jnp.int32)

    for b in range(_L):
        for c in range(_E // _L):
            hist_v[b, pl.ds(c * _L, _L)] = jnp.zeros((_L,), jnp.float32)

    def chunk_body(ci, carry):
        cbase = tok0 + ci * _CHUNK
        pltpu.sync_copy(logits_hbm.at[pl.ds(cbase, _CHUNK)], in_v)

        def group_body(g, carry2):
            rows = g * _L + lane

            def exp_body(e, ks):
                v = plsc.load_gather(in_v, [rows, zeros_i + e])
                vb = plsc.bitcast(v, jnp.int32)
                kv = plsc.bitcast((vb & jnp.int32(~63)) | e, jnp.float32)
                m = [kv > ks[j] for j in range(_K - 1)]
                mx = [jnp.maximum(kv, ks[j]) for j in range(_K)]
                nk = [mx[0]] + [jnp.where(m[j - 1], ks[j - 1], mx[j])
                                for j in range(1, _K)]
                return tuple(nk)

            ks = lax.fori_loop(
                0, _E, exp_body, tuple([neg_inf] * _K), unroll=4)

            ids = [plsc.bitcast(ks[j], jnp.int32) & 63 for j in range(_K)]
            vs = [plsc.load_gather(in_v, [rows, ids[j]]) for j in range(_K)]

            es = [jnp.exp(vs[j] - vs[0]) for j in range(_K)]
            s = es[0]
            for j in range(1, _K):
                s = s + es[j]
            r = 1.0 / s

            for j in range(_K):
                col = zeros_i + j
                plsc.store_scatter(w_stage, [rows, col], es[j] * r)
                plsc.store_scatter(id_stage, [rows, col], ids[j])
                plsc.addupdate_scatter(hist_v, [lane, ids[j]], ones)
            return carry2

        lax.fori_loop(0, _CHUNK // _L, group_body, 0)
        pltpu.sync_copy(w_stage, w_hbm.at[pl.ds(cbase, _CHUNK)])
        pltpu.sync_copy(id_stage, id_hbm.at[pl.ds(cbase, _CHUNK)])
        return carry

    lax.fori_loop(0, _NCHUNK, chunk_body, 0)

    for c in range(_E // _L):
        acc = hist_v[0, pl.ds(c * _L, _L)]
        for rr in range(1, _L):
            acc = acc + hist_v[rr, pl.ds(c * _L, _L)]
        hist_red[pl.ds(c * _L, _L)] = acc
    pltpu.sync_copy(hist_red, hist_hbm.at[wid])


_router = functools.partial(
    pl.kernel,
    out_type=(
        jax.ShapeDtypeStruct((_T, _K), jnp.float32),
        jax.ShapeDtypeStruct((_T, _K), jnp.int32),
        jax.ShapeDtypeStruct((_NW, _E), jnp.float32),
    ),
    mesh=plsc.VectorSubcoreMesh(core_axis_name="c", subcore_axis_name="s"),
    compiler_params=pltpu.CompilerParams(needs_layout_passes=False),
    scratch_types=[
        pltpu.VMEM((_CHUNK, _E), jnp.float32),
        pltpu.VMEM((_CHUNK, _K), jnp.float32),
        pltpu.VMEM((_CHUNK, _K), jnp.int32),
        pltpu.VMEM((_L, _E), jnp.float32),
        pltpu.VMEM((_E,), jnp.float32),
    ],
)(_router_body)


def _hist_reduce_body(p_ref, o_ref):
    o_ref[...] = jnp.sum(p_ref[...], axis=0, keepdims=True)


def _hist_reduce(partials):
    out = pl.pallas_call(
        _hist_reduce_body,
        out_shape=jax.ShapeDtypeStruct((1, _E), jnp.float32),
    )(partials)
    return out.reshape(_E)


@jax.jit
def kernel(logits):
    topk_w, topk_ids, partials = _router(logits)
    tokens_per_expert = _hist_reduce(partials)
    return (logits, topk_w, topk_ids, tokens_per_expert)
